# baseline (speedup 1.0000x reference)
import functools

import jax
import jax.numpy as jnp
from jax import lax
from jax.experimental import pallas as pl
from jax.experimental.pallas import tpu as pltpu
from jax.experimental.pallas import tpu_sc as plsc


def _matmul(features, W):
    n, d_in = features.shape
    d_out = W.shape[1]
    blk = 1000
    assert n % blk == 0

    def body(x_ref, w_ref, o_ref):
        o_ref[...] = jnp.dot(x_ref[...], w_ref[...],
                             preferred_element_type=jnp.float32)

    return pl.pallas_call(
        body,
        grid=(n // blk,),
        in_specs=[
            pl.BlockSpec((blk, d_in), lambda i: (i, 0)),
            pl.BlockSpec((d_in, d_out), lambda i: (0, 0)),
        ],
        out_specs=pl.BlockSpec((blk, d_out), lambda i: (i, 0)),
        out_shape=jax.ShapeDtypeStruct((n, d_out), jnp.float32),
    )(features, W)


def _combine_relu(partials):
    _, n, d = partials.shape
    blk = 1000
    assert n % blk == 0

    def body(p_ref, o_ref):
        o_ref[...] = jnp.maximum(p_ref[0] + p_ref[1], 0.0)

    return pl.pallas_call(
        body,
        grid=(n // blk,),
        in_specs=[pl.BlockSpec((2, blk, d), lambda i: (0, i, 0))],
        out_specs=pl.BlockSpec((blk, d), lambda i: (i, 0)),
        out_shape=jax.ShapeDtypeStruct((n, d), jnp.float32),
    )(partials)


def _sc_edge_aggregate(support, src, dst, w):
    n, d = support.shape
    e = src.shape[0]
    info = plsc.get_sparse_core_info()
    nc, ns = info.num_cores, info.num_subcores
    nw = nc * ns
    ch = 64
    NB = 4
    NM = 8
    nchunk = -(-e // (nw * ch))
    nchunk = -(-nchunk // NM) * NM
    assert nchunk >= 2 * NM
    e_pad = nw * nchunk * ch
    rows_per_tile = n // ns
    assert rows_per_tile * ns == n
    dump = (rows_per_tile // 8) * 8
    dump_tail = n - ns * dump

    pad = e_pad - e
    src3 = jnp.concatenate(
        [src, jnp.zeros((pad,), jnp.int32)]).reshape(nw, nchunk, ch)
    dst3 = jnp.concatenate(
        [dst, jnp.zeros((pad,), jnp.int32)]).reshape(nw, nchunk, ch)
    w3 = jnp.concatenate(
        [w, jnp.zeros((pad,), jnp.float32)]).reshape(nw, nchunk, ch)

    mesh = plsc.VectorSubcoreMesh(core_axis_name="c", subcore_axis_name="s")

    scratch = [
        pltpu.VMEM((NM, ch), jnp.int32),
        pltpu.VMEM((NM, ch), jnp.int32),
        pltpu.VMEM((NM, ch), jnp.float32),
    ]
    scratch += [pltpu.VMEM((ch, d), jnp.float32) for _ in range(NB)]
    scratch += [pltpu.VMEM_SHARED((n, d), jnp.float32)]
    scratch += [pltpu.SemaphoreType.DMA for _ in range(NM + 2 * NB)]

    @functools.partial(
        pl.kernel,
        out_type=jax.ShapeDtypeStruct((nc, n, d), jnp.float32),
        mesh=mesh,
        scratch_types=scratch,
    )
    def edge_kernel(sup_hbm, src_hbm, dst_hbm, w_hbm, out_hbm, *sc):
        srcr, dstr, wr = sc[0], sc[1], sc[2]
        rows = sc[3:3 + NB]
        acc = sc[3 + NB]
        sm = sc[4 + NB:4 + NB + NM]
        sg = sc[4 + NB + NM:4 + NB + NM + NB]
        ss = sc[4 + 2 * NB + NM:4 + 3 * NB + NM]
        cid = lax.axis_index("c")
        sid = lax.axis_index("s")
        wid = sid * nc + cid

        def meta_start(j, m):
            pltpu.async_copy(src_hbm.at[wid, j], srcr.at[m], sm[m])
            pltpu.async_copy(dst_hbm.at[wid, j], dstr.at[m], sm[m])
            pltpu.async_copy(w_hbm.at[wid, j], wr.at[m], sm[m])

        def meta_wait(j, m):
            pltpu.make_async_copy(src_hbm.at[wid, j], srcr.at[m], sm[m]).wait()
            pltpu.make_async_copy(dst_hbm.at[wid, j], dstr.at[m], sm[m]).wait()
            pltpu.make_async_copy(w_hbm.at[wid, j], wr.at[m], sm[m]).wait()

        def gather_start(m, k):
            pltpu.async_copy(sup_hbm.at[srcr.at[m]], rows[k], sg[k])

        def gather_wait(m, k):
            pltpu.make_async_copy(sup_hbm.at[srcr.at[m]], rows[k], sg[k]).wait()

        def scatter_start(m, k):
            pltpu.async_copy(rows[k], acc.at[dstr.at[m]], ss[k], add=True)

        def scatter_wait(m, k):
            pltpu.make_async_copy(rows[k], acc.at[dstr.at[m]], ss[k]).wait()

        def scale(m, k):
            buf = rows[k]

            def grp(g, carry):
                w16 = wr[m, pl.ds(g * 16, 16)]
                for u in range(16):
                    ee = g * 16 + u
                    wb = w16[u]
                    for col in range(d // 16):
                        sl = pl.ds(col * 16, 16)
                        buf[ee, sl] = buf[ee, sl] * wb
                return carry
            lax.fori_loop(0, ch // 16, grp, 0)

        for m in range(NB):
            meta_start(m, m)

        def zrow(i, carry):
            for g in range(d // 16):
                rows[0][i, pl.ds(g * 16, 16)] = jnp.zeros((16,), jnp.float32)
            return carry
        lax.fori_loop(0, ch, zrow, 0)

        r0 = sid * rows_per_tile
        nfull = rows_per_tile // ch
        for q in range(nfull):
            pltpu.sync_copy(rows[0], acc.at[pl.ds(r0 + q * ch, ch)])
        rem = rows_per_tile - nfull * ch
        if rem:
            pltpu.sync_copy(rows[0].at[pl.ds(0, rem)],
                            acc.at[pl.ds(r0 + nfull * ch, rem)])

        meta_wait(0, 0)
        gather_start(0, 0)
        meta_wait(1, 1)
        gather_start(1, 1)
        plsc.subcore_barrier()

        def loop_body(g, carry):
            for m in range(NM):
                x = g * NM + m
                k = m % NB

                @pl.when(x >= 2)
                def _():
                    scatter_wait((m - 2) % NM, (m - 2) % NB)

                @pl.when(x + 2 < nchunk)
                def _():
                    meta_wait(x + 2, (m + 2) % NM)
                    gather_start((m + 2) % NM, (m + 2) % NB)
                gather_wait(m, k)
                scale(m, k)
                scatter_start(m, k)

                @pl.when(x + 4 < nchunk)
                def _():
                    meta_start(x + 4, (m + 4) % NM)
            return carry
        lax.fori_loop(0, nchunk // NM, loop_body, 0)

        jl = nchunk - 1
        scatter_wait((jl - 1) % NM, (jl - 1) % NB)
        scatter_wait(jl % NM, jl % NB)

        plsc.subcore_barrier()
        pltpu.sync_copy(acc.at[pl.ds(sid * dump, dump)],
                        out_hbm.at[cid, pl.ds(sid * dump, dump)])

        @pl.when(sid == ns - 1)
        def _():
            if dump_tail:
                pltpu.sync_copy(acc.at[pl.ds(ns * dump, dump_tail)],
                                out_hbm.at[cid, pl.ds(ns * dump, dump_tail)])

    return edge_kernel(support, src3, dst3, w3)


def kernel(features, edge_index, edge_weight, W):
    support = _matmul(features, W)
    src = edge_index[0]
    dst = edge_index[1]
    partials = _sc_edge_aggregate(support, src, dst, edge_weight)
    return _combine_relu(partials)


# speedup vs baseline: 1.2216x; 1.2216x over previous
import functools

import jax
import jax.numpy as jnp
import numpy as np
from jax import lax
from jax.experimental import pallas as pl
from jax.experimental.pallas import tpu as pltpu
from jax.experimental.pallas import tpu_sc as plsc


def _interleave_perm(d):
    within = np.empty(32, np.int32)
    within[0::2] = np.arange(16)
    within[1::2] = 16 + np.arange(16)
    blocks = np.arange(d // 32)[:, None] * 32 + within[None, :]
    return blocks.reshape(-1)


def _matmul_bf16(features, W):
    n, d_in = features.shape
    d_out = W.shape[1]
    blk = 1000
    assert n % blk == 0

    def body(x_ref, w_ref, o_ref):
        o_ref[...] = jnp.dot(x_ref[...], w_ref[...],
                             preferred_element_type=jnp.float32
                             ).astype(jnp.bfloat16)

    return pl.pallas_call(
        body,
        grid=(n // blk,),
        in_specs=[
            pl.BlockSpec((blk, d_in), lambda i: (i, 0)),
            pl.BlockSpec((d_in, d_out), lambda i: (0, 0)),
        ],
        out_specs=pl.BlockSpec((blk, d_out), lambda i: (i, 0)),
        out_shape=jax.ShapeDtypeStruct((n, d_out), jnp.bfloat16),
    )(features, W)


def _combine_relu(partials):
    _, n, d = partials.shape
    blk = 1000
    assert n % blk == 0

    def body(p_ref, o_ref):
        o_ref[...] = jnp.maximum(p_ref[0] + p_ref[1], 0.0)

    return pl.pallas_call(
        body,
        grid=(n // blk,),
        in_specs=[pl.BlockSpec((2, blk, d), lambda i: (0, i, 0))],
        out_specs=pl.BlockSpec((blk, d), lambda i: (i, 0)),
        out_shape=jax.ShapeDtypeStruct((n, d), jnp.float32),
    )(partials)


def _sc_edge_aggregate(support_bf16, src, dst, w):
    n, d2 = support_bf16.shape
    d = d2 * 2
    e = src.shape[0]
    info = plsc.get_sparse_core_info()
    nc, ns = info.num_cores, info.num_subcores
    nw = nc * ns
    ch = 96
    nchunk = -(-e // (nw * ch))
    if nchunk % 2 == 0:
        nchunk += 1
    assert nchunk >= 3
    e_pad = nw * nchunk * ch
    rows_per_tile = n // ns
    assert rows_per_tile * ns == n
    dump = (rows_per_tile // 8) * 8
    dump_tail = n - ns * dump

    pad = e_pad - e
    src3 = jnp.concatenate(
        [src, jnp.zeros((pad,), jnp.int32)]).reshape(nw, nchunk, ch)
    dst3 = jnp.concatenate(
        [dst, jnp.zeros((pad,), jnp.int32)]).reshape(nw, nchunk, ch)
    w3 = jnp.concatenate(
        [w, jnp.zeros((pad,), jnp.float32)]).reshape(nw, nchunk, ch)

    mesh = plsc.VectorSubcoreMesh(core_axis_name="c", subcore_axis_name="s")

    @functools.partial(
        pl.kernel,
        out_type=jax.ShapeDtypeStruct((nc, n, d), jnp.float32),
        mesh=mesh,
        compiler_params=pltpu.CompilerParams(needs_layout_passes=False,
                                             use_tc_tiling_on_sc=False),
        scratch_types=[
            pltpu.VMEM((nchunk, ch), jnp.int32),
            pltpu.VMEM((2, ch), jnp.int32),
            pltpu.VMEM((2, ch), jnp.float32),
            pltpu.VMEM((ch, d // 2), jnp.int32),
            pltpu.VMEM((ch, d // 2), jnp.int32),
            pltpu.VMEM((ch, d), jnp.float32),
            pltpu.VMEM((ch, d), jnp.float32),
            pltpu.VMEM_SHARED((n, d), jnp.float32),
            pltpu.SemaphoreType.DMA,
            pltpu.SemaphoreType.DMA,
            pltpu.SemaphoreType.DMA,
            pltpu.SemaphoreType.DMA,
            pltpu.SemaphoreType.DMA,
            pltpu.SemaphoreType.DMA,
        ],
    )
    def edge_kernel(sup_hbm, src_hbm, dst_hbm, w_hbm, out_hbm,
                    dst_v, srcr, wr, gb_a, gb_b, fb_a, fb_b, acc,
                    sm0, sm1, sg_a, sg_b, ss_a, ss_b):
        cid = lax.axis_index("c")
        sid = lax.axis_index("s")
        wid = sid * nc + cid
        sm = (sm0, sm1)
        gbufs = (gb_a, gb_b)
        fbufs = (fb_a, fb_b)
        sg = (sg_a, sg_b)
        ss = (ss_a, ss_b)

        def meta_start(j, k):
            pltpu.async_copy(src_hbm.at[wid, j], srcr.at[k], sm[k])
            pltpu.async_copy(w_hbm.at[wid, j], wr.at[k], sm[k])

        def meta_wait(j, k):
            pltpu.make_async_copy(src_hbm.at[wid, j], srcr.at[k], sm[k]).wait()
            pltpu.make_async_copy(w_hbm.at[wid, j], wr.at[k], sm[k]).wait()

        def gather_start(k):
            pltpu.async_copy(sup_hbm.at[srcr.at[k]], gbufs[k], sg[k])

        def gather_wait(k):
            pltpu.make_async_copy(
                sup_hbm.at[srcr.at[k]], gbufs[k], sg[k]).wait()

        def scatter_start(j, k):
            pltpu.async_copy(fbufs[k], acc.at[dst_v.at[j]], ss[k], add=True)

        def scatter_wait(j, k):
            pltpu.make_async_copy(fbufs[k], acc.at[dst_v.at[j]], ss[k]).wait()

        def scale(k):
            gbuf, fbuf = gbufs[k], fbufs[k]

            def grp(g, carry):
                w16 = wr[k, pl.ds(g * 16, 16)]
                for u in range(16):
                    ee = g * 16 + u
                    wb = w16[u]
                    for h in range(d // 32):
                        vi = gbuf[ee, pl.ds(h * 16, 16)]
                        lo = plsc.bitcast(vi << 16, jnp.float32)
                        hi = plsc.bitcast(vi & jnp.int32(-65536),
                                          jnp.float32)
                        fbuf[ee, pl.ds(h * 32, 16)] = lo * wb
                        fbuf[ee, pl.ds(h * 32 + 16, 16)] = hi * wb
                return carry
            lax.fori_loop(0, ch // 16, grp, 0)

        meta_start(0, 0)
        meta_start(1, 1)
        pltpu.sync_copy(dst_hbm.at[wid], dst_v)

        def zrow(i, carry):
            for g in range(d // 16):
                fb_a[i, pl.ds(g * 16, 16)] = jnp.zeros((16,), jnp.float32)
            return carry
        lax.fori_loop(0, ch, zrow, 0)

        r0 = sid * rows_per_tile
        nfull = rows_per_tile // ch
        for k in range(nfull):
            pltpu.sync_copy(fb_a, acc.at[pl.ds(r0 + k * ch, ch)])
        rem = rows_per_tile - nfull * ch
        if rem:
            pltpu.sync_copy(fb_a.at[pl.ds(0, rem)],
                            acc.at[pl.ds(r0 + nfull * ch, rem)])

        meta_wait(0, 0)
        gather_start(0)
        plsc.subcore_barrier()

        def pair_body(g, carry):
            j = g * 2

            @pl.when(j > 0)
            def _():
                scatter_wait(j - 1, 1)
            meta_wait(j + 1, 1)
            gather_start(1)
            gather_wait(0)
            scale(0)
            scatter_start(j, 0)

            @pl.when(j + 2 < nchunk)
            def _():
                meta_start(j + 2, 0)
            gather_wait(1)
            scale(1)
            scatter_start(j + 1, 1)

            @pl.when(j + 3 < nchunk)
            def _():
                meta_start(j + 3, 1)
            scatter_wait(j, 0)

            @pl.when(j + 2 < nchunk)
            def _():
                meta_wait(j + 2, 0)
                gather_start(0)
            return carry
        lax.fori_loop(0, (nchunk - 1) // 2, pair_body, 0)

        jl = nchunk - 1
        scatter_wait(jl - 1, 1)
        gather_wait(0)
        scale(0)
        scatter_start(jl, 0)
        scatter_wait(jl, 0)

        plsc.subcore_barrier()
        pltpu.sync_copy(acc.at[pl.ds(sid * dump, dump)],
                        out_hbm.at[cid, pl.ds(sid * dump, dump)])

        @pl.when(sid == ns - 1)
        def _():
            if dump_tail:
                pltpu.sync_copy(acc.at[pl.ds(ns * dump, dump_tail)],
                                out_hbm.at[cid, pl.ds(ns * dump, dump_tail)])

    return edge_kernel(support_bf16, src3, dst3, w3)


def kernel(features, edge_index, edge_weight, W):
    perm = _interleave_perm(W.shape[1])
    support_bf16 = _matmul_bf16(features, W[:, perm])
    n, d = support_bf16.shape
    support_i32 = jax.lax.bitcast_convert_type(
        support_bf16.reshape(n, d // 2, 2), jnp.int32)
    src = edge_index[0]
    dst = edge_index[1]
    partials = _sc_edge_aggregate(support_i32, src, dst, edge_weight)
    return _combine_relu(partials)
